# baseline (speedup 1.0000x reference)
import functools

import jax
import jax.numpy as jnp
from jax import lax
from jax.experimental import pallas as pl
from jax.experimental.pallas import tpu as pltpu
from jax.experimental.pallas import tpu_sc as plsc

N = 100000
E = 3200000
D = 16
NCORE = 2
NSUB = 16
NW = NCORE * NSUB
CH = 128
K = 4
NBUF = 2
ROWS = E // CH
RPW = ROWS // NW
G2 = RPW // (2 * K)
TK = RPW - G2 * 2 * K
XTRA = ROWS - RPW * NW
NPAD = 100352
RPT = NPAD // NSUB
AR = NPAD // 8
XR = N // 8
BLK = 512
GRID = (AR + BLK - 1) // BLK


def _make_sc(with_deg):
    mesh = plsc.VectorSubcoreMesh(core_axis_name="c", subcore_axis_name="s")
    agg_t = jax.ShapeDtypeStruct((NPAD, D), jnp.float32)
    deg_t = jax.ShapeDtypeStruct((NCORE, NPAD), jnp.float32)
    out_type = (agg_t, agg_t, deg_t) if with_deg else (agg_t, agg_t)

    scratch = [
        pltpu.VMEM_SHARED((NPAD, D), jnp.float32),
        pltpu.VMEM((NBUF, K, CH), jnp.int32),
        pltpu.VMEM((NBUF, K, CH), jnp.int32),
        pltpu.VMEM((NBUF, K * CH, D), jnp.float32),
        pltpu.SemaphoreType.DMA,
        pltpu.SemaphoreType.DMA((NBUF,)),
    ]
    if with_deg:
        scratch += [
            pltpu.VMEM_SHARED((NPAD,), jnp.float32),
            pltpu.VMEM((CH,), jnp.float32),
        ]

    def body(*args):
        if with_deg:
            (xp_hbm, src_hbm, dst_hbm, z2_hbm, z1_hbm, a0_out, a1_out,
             d_out,
             agg_sh, srci, dsti, rows, gsem, ssem, deg_sh, ones_v) = args
        else:
            (xp_hbm, src_hbm, dst_hbm, z2_hbm, a0_out, a1_out,
             agg_sh, srci, dsti, rows, gsem, ssem) = args
        c = lax.axis_index("c")
        s = lax.axis_index("s")
        x_rows = xp_hbm
        t0 = s * RPT
        pltpu.sync_copy(z2_hbm.at[pl.ds(t0, RPT)], agg_sh.at[pl.ds(t0, RPT)])
        if with_deg:
            pltpu.sync_copy(z1_hbm.at[pl.ds(t0, RPT)], deg_sh.at[pl.ds(t0, RPT)])
            for i in range(CH // 16):
                ones_v[pl.ds(i * 16, 16)] = jnp.ones((16,), jnp.float32)
        plsc.subcore_barrier()

        wid = c * NSUB + s
        row0 = wid * RPW

        def drain(b):
            pltpu.make_async_copy(z2_hbm.at[pl.ds(0, K * CH)], rows.at[b],
                                  ssem.at[b]).wait()
            if with_deg:
                pltpu.make_async_copy(src_hbm.at[pl.ds(0, K)], dsti.at[b],
                                      ssem.at[b]).wait()

        def fire_group(row, b, nchunk):
            pltpu.sync_copy(src_hbm.at[pl.ds(row, nchunk)],
                            srci.at[b, pl.ds(0, nchunk)])
            pltpu.sync_copy(dst_hbm.at[pl.ds(row, nchunk)],
                            dsti.at[b, pl.ds(0, nchunk)])
            gh = [pltpu.async_copy(x_rows.at[srci.at[b, j]],
                                   rows.at[b, pl.ds(j * CH, CH)], gsem)
                  for j in range(nchunk)]
            for h in gh:
                h.wait()
            for j in range(nchunk):
                pltpu.async_copy(rows.at[b, pl.ds(j * CH, CH)],
                                 agg_sh.at[dsti.at[b, j]], ssem.at[b],
                                 add=True)
                if with_deg:
                    pltpu.async_copy(ones_v, deg_sh.at[dsti.at[b, j]],
                                     ssem.at[b], add=True)

        def dbl_group(g2, carry):
            for b in range(NBUF):
                @pl.when(g2 >= 1)
                def _(b=b):
                    drain(b)
                fire_group(row0 + (NBUF * g2 + b) * K, b, K)
            return carry

        lax.fori_loop(0, G2, dbl_group, 0)
        drain(0)
        drain(1)

        trow = row0 + G2 * NBUF * K
        fire_group(trow, 0, K)
        fire_group(trow + K, 1, TK - K)
        drain(0)

        @pl.when(wid < XTRA)
        def _():
            fire_group(NW * RPW + wid, 0, 1)

        for j in range(TK - K):
            pltpu.make_async_copy(z2_hbm.at[pl.ds(0, CH)],
                                  rows.at[1, pl.ds(0, CH)], ssem.at[1]).wait()
            if with_deg:
                pltpu.make_async_copy(src_hbm.at[pl.ds(0, 1)],
                                      dsti.at[1, pl.ds(0, 1)],
                                      ssem.at[1]).wait()

        @pl.when(wid < XTRA)
        def _():
            pltpu.make_async_copy(z2_hbm.at[pl.ds(0, CH)],
                                  rows.at[0, pl.ds(0, CH)], ssem.at[0]).wait()
            if with_deg:
                pltpu.make_async_copy(src_hbm.at[pl.ds(0, 1)],
                                      dsti.at[0, pl.ds(0, 1)],
                                      ssem.at[0]).wait()

        plsc.subcore_barrier()
        aggv = agg_sh.at[pl.ds(t0, RPT)]
        if with_deg:
            pltpu.sync_copy(deg_sh.at[pl.ds(t0, RPT)],
                            d_out.at[c, pl.ds(t0, RPT)])

        @pl.when(c == 0)
        def _():
            pltpu.sync_copy(aggv, a0_out.at[pl.ds(t0, RPT)])

        @pl.when(c == 1)
        def _():
            pltpu.sync_copy(aggv, a1_out.at[pl.ds(t0, RPT)])

    return pl.kernel(body, out_type=out_type, mesh=mesh,
                     scratch_types=scratch,
                     compiler_params=pltpu.CompilerParams(
                         use_tc_tiling_on_sc=False))


_sc_deg = _make_sc(True)
_sc_nodeg = _make_sc(False)


def _tc_body(x_ref, a0_ref, a1_ref, d0_ref, d1_ref, s_ref, w1_ref, w2_ref,
             b_ref, o_ref):
    agg = a0_ref[...] + a1_ref[...]
    deg = jnp.maximum(d0_ref[...] + d1_ref[...], 1.0)
    dot = functools.partial(jnp.dot, preferred_element_type=jnp.float32,
                            precision=lax.Precision.HIGHEST)
    dpk = dot(1.0 / deg, s_ref[...])
    o_ref[...] = (dot(agg * dpk, w1_ref[...]) + dot(x_ref[...], w2_ref[...])
                  + b_ref[...])


_tc = pl.pallas_call(
    _tc_body,
    grid=(GRID,),
    in_specs=[
        pl.BlockSpec((BLK, 128), lambda i: (i, 0)),
        pl.BlockSpec((BLK, 128), lambda i: (i, 0)),
        pl.BlockSpec((BLK, 128), lambda i: (i, 0)),
        pl.BlockSpec((BLK, 8), lambda i: (i, 0)),
        pl.BlockSpec((BLK, 8), lambda i: (i, 0)),
        pl.BlockSpec((8, 128), lambda i: (0, 0)),
        pl.BlockSpec((128, 128), lambda i: (0, 0)),
        pl.BlockSpec((128, 128), lambda i: (0, 0)),
        pl.BlockSpec((1, 128), lambda i: (0, 0)),
    ],
    out_specs=pl.BlockSpec((BLK, 128), lambda i: (i, 0)),
    out_shape=jax.ShapeDtypeStruct((AR, 128), jnp.float32),
)


def kernel(x, edge_index, Wl, bl, Wr, Wlin, blin):
    L = Wl.shape[0]
    z2 = jnp.zeros((NPAD, D), jnp.float32)
    z1 = jnp.zeros((NPAD,), jnp.float32)

    eye8 = jnp.eye(8, dtype=jnp.float32)
    S = jnp.kron(eye8, jnp.ones((1, 16), jnp.float32))
    w1b = [jnp.kron(eye8, Wl[l].T) for l in range(L)]
    w2b = [jnp.kron(eye8, (Wr[l] + Wlin[l]).T) for l in range(L)]
    bpk = [jnp.tile(bl[l] + blin[l], 8).reshape(1, 128) for l in range(L)]

    xp = jnp.pad(x.reshape(XR, 128), ((0, AR - XR), (0, 0)))
    srcr = edge_index[0].reshape(ROWS, CH)
    dstr = edge_index[1].reshape(ROWS, CH)
    d0 = d1 = None
    for l in range(L):
        xrows = xp.reshape(NPAD, D)
        if l == 0:
            a0, a1, degp = _sc_deg(xrows, srcr, dstr, z2, z1)
            d0 = degp[0].reshape(AR, 8)
            d1 = degp[1].reshape(AR, 8)
        else:
            a0, a1 = _sc_nodeg(xrows, srcr, dstr, z2)
        xp = _tc(xp, a0.reshape(AR, 128), a1.reshape(AR, 128), d0, d1, S,
                 w1b[l], w2b[l], bpk[l])
    return xp[:XR].reshape(N, D)


# speedup vs baseline: 1.0011x; 1.0011x over previous
import functools

import jax
import jax.numpy as jnp
from jax import lax
from jax.experimental import pallas as pl
from jax.experimental.pallas import tpu as pltpu
from jax.experimental.pallas import tpu_sc as plsc

N = 100000
E = 3200000
D = 16
NCORE = 2
NSUB = 16
NW = NCORE * NSUB
CH = 128
K = 4
NBUF = 2
ROWS = E // CH
RPW = ROWS // NW
G2 = RPW // (2 * K)
TK = RPW - (G2 * 2 + 1) * K
XTRA = ROWS - RPW * NW
NPAD = 100352
RPT = NPAD // NSUB
AR = NPAD // 8
XR = N // 8
BLK = 512
GRID = (AR + BLK - 1) // BLK


def _make_sc(with_deg):
    mesh = plsc.VectorSubcoreMesh(core_axis_name="c", subcore_axis_name="s")
    agg_t = jax.ShapeDtypeStruct((NPAD, D), jnp.float32)
    deg_t = jax.ShapeDtypeStruct((NCORE, NPAD), jnp.float32)
    out_type = (agg_t, agg_t, deg_t) if with_deg else (agg_t, agg_t)

    scratch = [
        pltpu.VMEM_SHARED((NPAD, D), jnp.float32),
        pltpu.VMEM((NBUF, K, CH), jnp.int32),
        pltpu.VMEM((NBUF, K, CH), jnp.int32),
        pltpu.VMEM((NBUF, K * CH, D), jnp.float32),
        pltpu.SemaphoreType.DMA,
        pltpu.SemaphoreType.DMA((NBUF,)),
    ]
    if with_deg:
        scratch += [
            pltpu.VMEM_SHARED((NPAD,), jnp.float32),
            pltpu.VMEM((CH,), jnp.float32),
        ]

    def body(*args):
        if with_deg:
            (xp_hbm, src_hbm, dst_hbm, z2_hbm, z1_hbm, a0_out, a1_out,
             d_out,
             agg_sh, srci, dsti, rows, gsem, ssem, deg_sh, ones_v) = args
        else:
            (xp_hbm, src_hbm, dst_hbm, z2_hbm, a0_out, a1_out,
             agg_sh, srci, dsti, rows, gsem, ssem) = args
        c = lax.axis_index("c")
        s = lax.axis_index("s")
        x_rows = xp_hbm
        t0 = s * RPT
        pltpu.sync_copy(z2_hbm.at[pl.ds(t0, RPT)], agg_sh.at[pl.ds(t0, RPT)])
        if with_deg:
            pltpu.sync_copy(z1_hbm.at[pl.ds(t0, RPT)], deg_sh.at[pl.ds(t0, RPT)])
            for i in range(CH // 16):
                ones_v[pl.ds(i * 16, 16)] = jnp.ones((16,), jnp.float32)
        plsc.subcore_barrier()

        wid = c * NSUB + s
        row0 = wid * RPW

        def drain_scat(b, nchunk):
            pltpu.make_async_copy(z2_hbm.at[pl.ds(0, nchunk * CH)],
                                  rows.at[b, pl.ds(0, nchunk * CH)],
                                  ssem.at[b]).wait()
            if with_deg:
                pltpu.make_async_copy(src_hbm.at[pl.ds(0, nchunk)],
                                      dsti.at[b, pl.ds(0, nchunk)],
                                      ssem.at[b]).wait()

        def wait_gath(b, nchunk):
            pltpu.make_async_copy(z2_hbm.at[pl.ds(0, nchunk * CH)],
                                  rows.at[b, pl.ds(0, nchunk * CH)],
                                  gsem).wait()

        def load_idx(row, b, nchunk):
            pltpu.sync_copy(src_hbm.at[pl.ds(row, nchunk)],
                            srci.at[b, pl.ds(0, nchunk)])
            pltpu.sync_copy(dst_hbm.at[pl.ds(row, nchunk)],
                            dsti.at[b, pl.ds(0, nchunk)])

        def fire_gath(b, nchunk):
            for j in range(nchunk):
                pltpu.async_copy(x_rows.at[srci.at[b, j]],
                                 rows.at[b, pl.ds(j * CH, CH)], gsem)

        def fire_scat(b, nchunk):
            for j in range(nchunk):
                pltpu.async_copy(rows.at[b, pl.ds(j * CH, CH)],
                                 agg_sh.at[dsti.at[b, j]], ssem.at[b],
                                 add=True)
                if with_deg:
                    pltpu.async_copy(ones_v, deg_sh.at[dsti.at[b, j]],
                                     ssem.at[b], add=True)

        load_idx(row0, 0, K)
        fire_gath(0, K)

        def half(g, g2, b):
            wait_gath(b, K)
            fire_scat(b, K)

            @pl.when(g2 + b >= 1)
            def _():
                drain_scat(1 - b, K)
            load_idx(row0 + (g + 1) * K, 1 - b, K)
            fire_gath(1 - b, K)

        def dbl_group(g2, carry):
            half(NBUF * g2, g2, 0)
            half(NBUF * g2 + 1, g2, 1)
            return carry

        lax.fori_loop(0, G2, dbl_group, 0)

        wait_gath(0, K)
        fire_scat(0, K)
        drain_scat(1, K)

        trow = row0 + (G2 * NBUF + 1) * K
        load_idx(trow, 1, TK)
        fire_gath(1, TK)
        wait_gath(1, TK)
        fire_scat(1, TK)
        drain_scat(1, TK)

        @pl.when(wid < XTRA)
        def _():
            load_idx(NW * RPW + wid, 1, 1)
            fire_gath(1, 1)
            wait_gath(1, 1)
            fire_scat(1, 1)
            drain_scat(1, 1)

        drain_scat(0, K)

        plsc.subcore_barrier()
        aggv = agg_sh.at[pl.ds(t0, RPT)]
        if with_deg:
            pltpu.sync_copy(deg_sh.at[pl.ds(t0, RPT)],
                            d_out.at[c, pl.ds(t0, RPT)])

        @pl.when(c == 0)
        def _():
            pltpu.sync_copy(aggv, a0_out.at[pl.ds(t0, RPT)])

        @pl.when(c == 1)
        def _():
            pltpu.sync_copy(aggv, a1_out.at[pl.ds(t0, RPT)])

    return pl.kernel(body, out_type=out_type, mesh=mesh,
                     scratch_types=scratch,
                     compiler_params=pltpu.CompilerParams(
                         use_tc_tiling_on_sc=False))


_sc_deg = _make_sc(True)
_sc_nodeg = _make_sc(False)


def _tc_body(x_ref, a0_ref, a1_ref, d0_ref, d1_ref, s_ref, w1_ref, w2_ref,
             b_ref, o_ref):
    agg = a0_ref[...] + a1_ref[...]
    deg = jnp.maximum(d0_ref[...] + d1_ref[...], 1.0)
    dot = functools.partial(jnp.dot, preferred_element_type=jnp.float32,
                            precision=lax.Precision.HIGHEST)
    dpk = dot(1.0 / deg, s_ref[...])
    o_ref[...] = (dot(agg * dpk, w1_ref[...]) + dot(x_ref[...], w2_ref[...])
                  + b_ref[...])


_tc = pl.pallas_call(
    _tc_body,
    grid=(GRID,),
    in_specs=[
        pl.BlockSpec((BLK, 128), lambda i: (i, 0)),
        pl.BlockSpec((BLK, 128), lambda i: (i, 0)),
        pl.BlockSpec((BLK, 128), lambda i: (i, 0)),
        pl.BlockSpec((BLK, 8), lambda i: (i, 0)),
        pl.BlockSpec((BLK, 8), lambda i: (i, 0)),
        pl.BlockSpec((8, 128), lambda i: (0, 0)),
        pl.BlockSpec((128, 128), lambda i: (0, 0)),
        pl.BlockSpec((128, 128), lambda i: (0, 0)),
        pl.BlockSpec((1, 128), lambda i: (0, 0)),
    ],
    out_specs=pl.BlockSpec((BLK, 128), lambda i: (i, 0)),
    out_shape=jax.ShapeDtypeStruct((AR, 128), jnp.float32),
)


def kernel(x, edge_index, Wl, bl, Wr, Wlin, blin):
    L = Wl.shape[0]
    z2 = jnp.zeros((NPAD, D), jnp.float32)
    z1 = jnp.zeros((NPAD,), jnp.float32)

    eye8 = jnp.eye(8, dtype=jnp.float32)
    S = jnp.kron(eye8, jnp.ones((1, 16), jnp.float32))
    w1b = [jnp.kron(eye8, Wl[l].T) for l in range(L)]
    w2b = [jnp.kron(eye8, (Wr[l] + Wlin[l]).T) for l in range(L)]
    bpk = [jnp.tile(bl[l] + blin[l], 8).reshape(1, 128) for l in range(L)]

    xp = jnp.pad(x.reshape(XR, 128), ((0, AR - XR), (0, 0)))
    srcr = edge_index[0].reshape(ROWS, CH)
    dstr = edge_index[1].reshape(ROWS, CH)
    d0 = d1 = None
    for l in range(L):
        xrows = xp.reshape(NPAD, D)
        if l == 0:
            a0, a1, degp = _sc_deg(xrows, srcr, dstr, z2, z1)
            d0 = degp[0].reshape(AR, 8)
            d1 = degp[1].reshape(AR, 8)
        else:
            a0, a1 = _sc_nodeg(xrows, srcr, dstr, z2)
        xp = _tc(xp, a0.reshape(AR, 128), a1.reshape(AR, 128), d0, d1, S,
                 w1b[l], w2b[l], bpk[l])
    return xp[:XR].reshape(N, D)


# speedup vs baseline: 1.5176x; 1.5159x over previous
import functools

import jax
import jax.numpy as jnp
from jax import lax
from jax.experimental import pallas as pl
from jax.experimental.pallas import tpu as pltpu
from jax.experimental.pallas import tpu_sc as plsc

N = 100000
E = 3200000
D = 16
NCORE = 2
NSUB = 16
NW = NCORE * NSUB
CH = 128
K = 4
NBUF = 2
NIDX = 3
ROWS = E // CH
RPW = ROWS // NW
NG = RPW // K
PEEL = 2
GL = (NG - 3) // 6
XTRA = ROWS - RPW * NW
NPAD = 100352
RPT = NPAD // NSUB
AR = NPAD // 8
XR = N // 8
BLK = 512
GRID = (AR + BLK - 1) // BLK


def _make_sc(with_deg):
    mesh = plsc.VectorSubcoreMesh(core_axis_name="c", subcore_axis_name="s")
    agg_t = jax.ShapeDtypeStruct((NPAD, D), jnp.float32)
    deg_t = jax.ShapeDtypeStruct((NCORE, NPAD), jnp.float32)
    out_type = (agg_t, agg_t, deg_t) if with_deg else (agg_t, agg_t)

    scratch = [
        pltpu.VMEM_SHARED((NPAD, D), jnp.float32),
        pltpu.VMEM((NIDX, K, CH), jnp.int32),
        pltpu.VMEM((NIDX, K, CH), jnp.int32),
        pltpu.VMEM((NBUF, K * CH, D), jnp.float32),
        pltpu.SemaphoreType.DMA,
        pltpu.SemaphoreType.DMA((NBUF,)),
        pltpu.SemaphoreType.DMA((NIDX,)),
    ]
    if with_deg:
        scratch += [
            pltpu.VMEM_SHARED((NPAD,), jnp.float32),
            pltpu.VMEM((CH,), jnp.float32),
        ]

    def body(*args):
        if with_deg:
            (xp_hbm, src_hbm, dst_hbm, z2_hbm, z1_hbm, a0_out, a1_out,
             d_out,
             agg_sh, srci, dsti, rows, gsem, ssem, isem,
             deg_sh, ones_v) = args
        else:
            (xp_hbm, src_hbm, dst_hbm, z2_hbm, a0_out, a1_out,
             agg_sh, srci, dsti, rows, gsem, ssem, isem) = args
        c = lax.axis_index("c")
        s = lax.axis_index("s")
        x_rows = xp_hbm
        t0 = s * RPT
        pltpu.sync_copy(z2_hbm.at[pl.ds(t0, RPT)], agg_sh.at[pl.ds(t0, RPT)])
        if with_deg:
            pltpu.sync_copy(z1_hbm.at[pl.ds(t0, RPT)], deg_sh.at[pl.ds(t0, RPT)])
            for i in range(CH // 16):
                ones_v[pl.ds(i * 16, 16)] = jnp.ones((16,), jnp.float32)
        plsc.subcore_barrier()

        wid = c * NSUB + s
        row0 = wid * RPW

        def drain_scat(r, nchunk):
            pltpu.make_async_copy(z2_hbm.at[pl.ds(0, nchunk * CH)],
                                  rows.at[r, pl.ds(0, nchunk * CH)],
                                  ssem.at[r]).wait()
            if with_deg:
                pltpu.make_async_copy(src_hbm.at[pl.ds(0, nchunk)],
                                      srci.at[0, pl.ds(0, nchunk)],
                                      ssem.at[r]).wait()

        def wait_gath(r, nchunk):
            pltpu.make_async_copy(z2_hbm.at[pl.ds(0, nchunk * CH)],
                                  rows.at[r, pl.ds(0, nchunk * CH)],
                                  gsem).wait()

        def load_idx_async(row, q, nchunk):
            pltpu.async_copy(src_hbm.at[pl.ds(row, nchunk)],
                             srci.at[q, pl.ds(0, nchunk)], isem.at[q])
            pltpu.async_copy(dst_hbm.at[pl.ds(row, nchunk)],
                             dsti.at[q, pl.ds(0, nchunk)], isem.at[q])

        def wait_idx(q, nchunk):
            pltpu.make_async_copy(src_hbm.at[pl.ds(0, nchunk)],
                                  srci.at[q, pl.ds(0, nchunk)],
                                  isem.at[q]).wait()
            pltpu.make_async_copy(dst_hbm.at[pl.ds(0, nchunk)],
                                  dsti.at[q, pl.ds(0, nchunk)],
                                  isem.at[q]).wait()

        def fire_gath(r, q, nchunk):
            for j in range(nchunk):
                pltpu.async_copy(x_rows.at[srci.at[q, j]],
                                 rows.at[r, pl.ds(j * CH, CH)], gsem)

        def fire_scat(r, q, nchunk):
            for j in range(nchunk):
                pltpu.async_copy(rows.at[r, pl.ds(j * CH, CH)],
                                 agg_sh.at[dsti.at[q, j]], ssem.at[r],
                                 add=True)
                if with_deg:
                    pltpu.async_copy(ones_v, deg_sh.at[dsti.at[q, j]],
                                     ssem.at[r], add=True)

        def body(g):
            r, q = g % 2, g % 3
            wait_gath(r, K)
            fire_scat(r, q, K)
            drain_scat(1 - r, K)
            load_idx_async(row0 + (g + 2) * K, (g + 2) % 3, K)
            wait_idx((g + 1) % 3, K)
            fire_gath(1 - r, (g + 1) % 3, K)

        load_idx_async(row0, 0, K)
        load_idx_async(row0 + K, 1, K)
        load_idx_async(row0 + 2 * K, 2, K)
        wait_idx(0, K)
        fire_gath(0, 0, K)
        wait_gath(0, K)
        fire_scat(0, 0, K)
        wait_idx(1, K)
        fire_gath(1, 1, K)

        def six_groups(i, carry):
            g0 = 1 + 6 * i
            for dg in range(6):
                body(g0 + dg)
            return carry

        lax.fori_loop(0, GL, six_groups, 0)

        body(193)
        wait_gath(0, K)
        fire_scat(0, 2, K)
        drain_scat(1, K)
        wait_idx(0, K)
        fire_gath(1, 0, 1)
        wait_gath(1, 1)
        fire_scat(1, 0, 1)
        drain_scat(0, K)

        @pl.when(wid < XTRA)
        def _():
            pltpu.sync_copy(src_hbm.at[pl.ds(NW * RPW + wid, 1)],
                            srci.at[1, pl.ds(0, 1)])
            pltpu.sync_copy(dst_hbm.at[pl.ds(NW * RPW + wid, 1)],
                            dsti.at[1, pl.ds(0, 1)])
            fire_gath(0, 1, 1)
            wait_gath(0, 1)
            fire_scat(0, 1, 1)
            drain_scat(0, 1)

        drain_scat(1, 1)

        plsc.subcore_barrier()
        aggv = agg_sh.at[pl.ds(t0, RPT)]
        if with_deg:
            pltpu.sync_copy(deg_sh.at[pl.ds(t0, RPT)],
                            d_out.at[c, pl.ds(t0, RPT)])

        @pl.when(c == 0)
        def _():
            pltpu.sync_copy(aggv, a0_out.at[pl.ds(t0, RPT)])

        @pl.when(c == 1)
        def _():
            pltpu.sync_copy(aggv, a1_out.at[pl.ds(t0, RPT)])

    return pl.kernel(body, out_type=out_type, mesh=mesh,
                     scratch_types=scratch,
                     compiler_params=pltpu.CompilerParams(
                         use_tc_tiling_on_sc=False))


_sc_deg = _make_sc(True)
_sc_nodeg = _make_sc(False)


def _tc_body(x_ref, a0_ref, a1_ref, d0_ref, d1_ref, s_ref, w1_ref, w2_ref,
             b_ref, o_ref):
    agg = a0_ref[...] + a1_ref[...]
    deg = jnp.maximum(d0_ref[...] + d1_ref[...], 1.0)
    dot = functools.partial(jnp.dot, preferred_element_type=jnp.float32,
                            precision=lax.Precision.HIGHEST)
    dpk = dot(1.0 / deg, s_ref[...])
    o_ref[...] = (dot(agg * dpk, w1_ref[...]) + dot(x_ref[...], w2_ref[...])
                  + b_ref[...])


_tc = pl.pallas_call(
    _tc_body,
    grid=(GRID,),
    in_specs=[
        pl.BlockSpec((BLK, 128), lambda i: (i, 0)),
        pl.BlockSpec((BLK, 128), lambda i: (i, 0)),
        pl.BlockSpec((BLK, 128), lambda i: (i, 0)),
        pl.BlockSpec((BLK, 8), lambda i: (i, 0)),
        pl.BlockSpec((BLK, 8), lambda i: (i, 0)),
        pl.BlockSpec((8, 128), lambda i: (0, 0)),
        pl.BlockSpec((128, 128), lambda i: (0, 0)),
        pl.BlockSpec((128, 128), lambda i: (0, 0)),
        pl.BlockSpec((1, 128), lambda i: (0, 0)),
    ],
    out_specs=pl.BlockSpec((BLK, 128), lambda i: (i, 0)),
    out_shape=jax.ShapeDtypeStruct((AR, 128), jnp.float32),
)


def kernel(x, edge_index, Wl, bl, Wr, Wlin, blin):
    L = Wl.shape[0]
    z2 = jnp.zeros((NPAD, D), jnp.float32)
    z1 = jnp.zeros((NPAD,), jnp.float32)

    eye8 = jnp.eye(8, dtype=jnp.float32)
    S = jnp.kron(eye8, jnp.ones((1, 16), jnp.float32))
    w1b = [jnp.kron(eye8, Wl[l].T) for l in range(L)]
    w2b = [jnp.kron(eye8, (Wr[l] + Wlin[l]).T) for l in range(L)]
    bpk = [jnp.tile(bl[l] + blin[l], 8).reshape(1, 128) for l in range(L)]

    xp = jnp.pad(x.reshape(XR, 128), ((0, AR - XR), (0, 0)))
    srcr = edge_index[0].reshape(ROWS, CH)
    dstr = edge_index[1].reshape(ROWS, CH)
    d0 = d1 = None
    for l in range(L):
        xrows = xp.reshape(NPAD, D)
        if l == 0:
            a0, a1, degp = _sc_deg(xrows, srcr, dstr, z2, z1)
            d0 = degp[0].reshape(AR, 8)
            d1 = degp[1].reshape(AR, 8)
        else:
            a0, a1 = _sc_nodeg(xrows, srcr, dstr, z2)
        xp = _tc(xp, a0.reshape(AR, 128), a1.reshape(AR, 128), d0, d1, S,
                 w1b[l], w2b[l], bpk[l])
    return xp[:XR].reshape(N, D)
